# Initial kernel scaffold; baseline (speedup 1.0000x reference)
import jax
import jax.numpy as jnp
from jax.experimental import pallas as pl


def kernel(feature, adj, W):
    raise NotImplementedError("write your pallas kernel here")


# speedup vs baseline: 3.5515x; 3.5515x over previous
import functools

import jax
import jax.numpy as jnp
from jax.experimental import pallas as pl
from jax.experimental.pallas import tpu as pltpu


def _support_body(f_ref, w_ref, out_ref):
    out_ref[...] = jnp.maximum(
        jnp.dot(f_ref[...], w_ref[...], preferred_element_type=jnp.float32), 0.0
    )


def _pass1_body(adj_ref, x_ref, s_ref, t_ref, low_ref):
    t = jnp.dot(adj_ref[...], x_ref[...], preferred_element_type=jnp.float32)
    t_ref[...] = t
    low_ref[...] = t + s_ref[...]


def _pass2_body(adj_ref, x_ref, s_ref, mid_ref):
    t = jnp.dot(adj_ref[...], x_ref[...], preferred_element_type=jnp.float32)
    mid_ref[...] = t - s_ref[...]


@functools.partial(jax.jit, static_argnames=())
def kernel(feature, adj, W):
    n, f_in = feature.shape
    d = W.shape[1]
    dtype = feature.dtype

    bm_s = 512
    support = pl.pallas_call(
        _support_body,
        grid=(n // bm_s,),
        in_specs=[
            pl.BlockSpec((bm_s, f_in), lambda i: (i, 0)),
            pl.BlockSpec((f_in, d), lambda i: (0, 0)),
        ],
        out_specs=pl.BlockSpec((bm_s, d), lambda i: (i, 0)),
        out_shape=jax.ShapeDtypeStruct((n, d), dtype),
        compiler_params=pltpu.CompilerParams(
            dimension_semantics=("arbitrary",)
        ),
    )(feature, W)

    bm = 256
    grid = (n // bm,)
    adj_spec = pl.BlockSpec((bm, n), lambda i: (i, 0))
    full_spec = pl.BlockSpec((n, d), lambda i: (0, 0))
    row_spec = pl.BlockSpec((bm, d), lambda i: (i, 0))
    row_shape = jax.ShapeDtypeStruct((n, d), dtype)
    params = pltpu.CompilerParams(dimension_semantics=("arbitrary",))

    t1, out_low = pl.pallas_call(
        _pass1_body,
        grid=grid,
        in_specs=[adj_spec, full_spec, row_spec],
        out_specs=[row_spec, row_spec],
        out_shape=[row_shape, row_shape],
        compiler_params=params,
    )(adj, support, support)

    out_mid = pl.pallas_call(
        _pass2_body,
        grid=grid,
        in_specs=[adj_spec, full_spec, row_spec],
        out_specs=row_spec,
        out_shape=row_shape,
        compiler_params=params,
    )(adj, t1, support)

    output = jnp.concatenate([out_low[:, None, :], out_mid[:, None, :]], axis=1)
    return (output, out_low, out_mid)


# speedup vs baseline: 4.6037x; 1.2962x over previous
import functools

import jax
import jax.numpy as jnp
from jax.experimental import pallas as pl
from jax.experimental.pallas import tpu as pltpu


def _support_body(f_ref, w_ref, out_ref):
    out_ref[...] = jnp.maximum(
        jnp.dot(f_ref[...], w_ref[...], preferred_element_type=jnp.float32), 0.0
    )


def _pass1_body(adj_ref, x_ref, s_ref, t_ref, low_ref):
    t = jnp.dot(adj_ref[...], x_ref[...], preferred_element_type=jnp.float32)
    t_ref[...] = t
    low_ref[...] = t + s_ref[...]


def _pass2_body(adj_ref, x_ref, s_ref, low_ref, mid_ref, cat_ref):
    t = jnp.dot(adj_ref[...], x_ref[...], preferred_element_type=jnp.float32)
    mid = t - s_ref[...]
    mid_ref[...] = mid
    cat_ref[:, 0, :] = low_ref[...]
    cat_ref[:, 1, :] = mid


@functools.partial(jax.jit, static_argnames=())
def kernel(feature, adj, W):
    n, f_in = feature.shape
    d = W.shape[1]
    dtype = feature.dtype

    bm_s = 512
    support = pl.pallas_call(
        _support_body,
        grid=(n // bm_s,),
        in_specs=[
            pl.BlockSpec((bm_s, f_in), lambda i: (i, 0)),
            pl.BlockSpec((f_in, d), lambda i: (0, 0)),
        ],
        out_specs=pl.BlockSpec((bm_s, d), lambda i: (i, 0)),
        out_shape=jax.ShapeDtypeStruct((n, d), dtype),
        compiler_params=pltpu.CompilerParams(
            dimension_semantics=("arbitrary",)
        ),
    )(feature, W)

    bm = 512
    grid = (n // bm,)
    adj_spec = pl.BlockSpec((bm, n), lambda i: (i, 0))
    full_spec = pl.BlockSpec((n, d), lambda i: (0, 0))
    row_spec = pl.BlockSpec((bm, d), lambda i: (i, 0))
    row_shape = jax.ShapeDtypeStruct((n, d), dtype)
    params = pltpu.CompilerParams(dimension_semantics=("arbitrary",))

    t1, out_low = pl.pallas_call(
        _pass1_body,
        grid=grid,
        in_specs=[adj_spec, full_spec, row_spec],
        out_specs=[row_spec, row_spec],
        out_shape=[row_shape, row_shape],
        compiler_params=params,
    )(adj, support, support)

    out_mid, output = pl.pallas_call(
        _pass2_body,
        grid=grid,
        in_specs=[adj_spec, full_spec, row_spec, row_spec],
        out_specs=[row_spec, pl.BlockSpec((bm, 2, d), lambda i: (i, 0, 0))],
        out_shape=[row_shape, jax.ShapeDtypeStruct((n, 2, d), dtype)],
        compiler_params=params,
    )(adj, t1, support, out_low)

    return (output, out_low, out_mid)
